# baseline (speedup 1.0000x reference)
import functools

import jax
import jax.numpy as jnp
from jax import lax
from jax.experimental import pallas as pl
from jax.experimental.pallas import tpu as pltpu
from jax.experimental.pallas import tpu_sc as plsc

BATCH = 16384
SEQ = 50
DIM = 64
NBUF = 8


def _build(num_workers: int):
  b_per_w = BATCH // num_workers
  mesh = plsc.VectorSubcoreMesh(core_axis_name="c", subcore_axis_name="s")
  nc = mesh.num_cores

  @functools.partial(
      pl.kernel,
      out_type=jax.ShapeDtypeStruct((BATCH, SEQ, DIM), jnp.float32),
      mesh=mesh,
      scratch_types=[
          pltpu.VMEM((b_per_w, SEQ), jnp.int32),
          pltpu.VMEM((NBUF, SEQ, DIM), jnp.float32),
          pltpu.SemaphoreType.DMA,
      ],
      compiler_params=pltpu.CompilerParams(use_tc_tiling_on_sc=False),
  )
  def gather_kernel(ids_hbm, table_hbm, out_hbm, idx_v, rows_v, sem):
    wid = lax.axis_index("s") * nc + lax.axis_index("c")
    base = wid * b_per_w
    pltpu.sync_copy(ids_hbm.at[pl.ds(base, b_per_w)], idx_v)

    def fire(r, slot):
      pltpu.async_copy(table_hbm.at[idx_v.at[r]], rows_v.at[slot], sem)

    def drain(slot):
      pltpu.make_async_copy(
          table_hbm.at[idx_v.at[0]], rows_v.at[slot], sem
      ).wait()

    for b in range(NBUF):
      fire(b, b)

    @pl.loop(0, b_per_w, step=NBUF)
    def _(r):
      for b in range(NBUF):
        g = r + b
        drain(b)
        pltpu.sync_copy(rows_v.at[b], out_hbm.at[base + g])

        @pl.when(g + NBUF < b_per_w)
        def _():
          fire(g + NBUF, b)

  return gather_kernel


def kernel(ids, table):
  return _build(32)(ids.astype(jnp.int32), table)
